# baseline (speedup 1.0000x reference)
import functools

import jax
import jax.numpy as jnp
from jax import lax
from jax.experimental import pallas as pl
from jax.experimental.pallas import tpu as pltpu, tpu_sc as plsc

_BEAM = 20
_PRE = 30
_PAD = 32
_L = 16
_NEG = float("-inf")


@functools.cache
def _build():
    return functools.partial(
        pl.kernel,
        mesh=plsc.VectorSubcoreMesh(
            core_axis_name="c", subcore_axis_name="s", num_cores=1),
        out_type=jax.ShapeDtypeStruct((2 * _BEAM,), jnp.int32),
        scratch_types=[
            pltpu.VMEM((_PAD,), jnp.int32),
            pltpu.VMEM((_PAD,), jnp.float32),
            pltpu.VMEM((2 * _BEAM,), jnp.int32),
            pltpu.SemaphoreType.DMA,
        ],
        compiler_params=pltpu.CompilerParams(needs_layout_passes=False),
    )(_beam_topk)


def _beam_topk(ws_hbm, ids_hbm, out_hbm, idx_v, vals_v, o_v, sem):
    is_worker = jnp.logical_and(
        lax.axis_index("c") == 0, lax.axis_index("s") == 0)

    @pl.when(is_worker)
    def _():
        idx_v[pl.ds(_L, _L)] = jnp.zeros((_L,), jnp.int32)
        pltpu.sync_copy(ids_hbm, idx_v.at[pl.ds(0, _PRE)])
        pltpu.async_copy(ws_hbm.at[idx_v], vals_v, sem).wait()

        lane = lax.iota(jnp.int32, _L)
        g0 = idx_v[pl.ds(0, _L)]
        g1 = idx_v[pl.ds(_L, _L)]
        v0 = vals_v[pl.ds(0, _L)]
        v1 = jnp.where(lane < _PRE - _L, vals_v[pl.ds(_L, _L)], _NEG)

        def bcast(a0, a1, j):
            src = a0 if j < _L else a1
            return jnp.take(src, jnp.full((_L,), j % _L, jnp.int32))

        false_v = jnp.zeros((_L,), jnp.bool_)
        dup0, dup1 = false_v, false_v
        for j in range(_PRE - 1):
            bg = bcast(g0, g1, j)
            if j < _L:
                dup0 = jnp.logical_or(
                    dup0, jnp.logical_and(g0 == bg, lane > j))
                dup1 = jnp.logical_or(dup1, g1 == bg)
            else:
                dup1 = jnp.logical_or(
                    dup1, jnp.logical_and(g1 == bg, lane > j - _L))
        neg = jnp.full((_L,), _NEG, jnp.float32)
        dval0 = jnp.where(dup0, neg, v0)
        dval1 = jnp.where(dup1, neg, v1)

        zero_v = jnp.zeros((_L,), jnp.int32)
        lrank0, lrank1 = zero_v, zero_v
        trank0, trank1 = zero_v, zero_v
        for j in range(_PRE):
            bv = bcast(v0, v1, j)
            bdv = bcast(dval0, dval1, j)
            bg = bcast(g0, g1, j)

            if j < _L:
                m0 = jnp.logical_or(
                    bv > v0, jnp.logical_and(bv == v0, lane > j))
                m1 = jnp.logical_or(bv > v1, bv == v1)
            else:
                m0 = bv > v0
                m1 = jnp.logical_or(
                    bv > v1, jnp.logical_and(bv == v1, lane > j - _L))
            lrank0 = lrank0 + m0.astype(jnp.int32)
            lrank1 = lrank1 + m1.astype(jnp.int32)

            t0 = jnp.logical_or(
                bdv > dval0, jnp.logical_and(bdv == dval0, bg < g0))
            t1 = jnp.logical_or(
                bdv > dval1, jnp.logical_and(bdv == dval1, bg < g1))
            trank0 = trank0 + t0.astype(jnp.int32)
            trank1 = trank1 + t1.astype(jnp.int32)

        beam1 = jnp.full((_L,), _BEAM - 1, jnp.int32)
        plsc.store_scatter(
            o_v, [jnp.minimum(trank0, beam1)], g0, mask=trank0 < _BEAM)
        plsc.store_scatter(
            o_v, [jnp.minimum(trank1, beam1)], g1, mask=trank1 < _BEAM)
        plsc.store_scatter(
            o_v, [_BEAM + jnp.minimum(lrank0, beam1)], lane,
            mask=lrank0 < _BEAM)
        plsc.store_scatter(
            o_v, [_BEAM + jnp.minimum(lrank1, beam1)], lane + _L,
            mask=lrank1 < _BEAM)

        pltpu.sync_copy(o_v, out_hbm)


def kernel(weighted_scores, ids):
    packed = _build()(weighted_scores, ids)
    return packed[:_BEAM], packed[_BEAM:]


# speedup vs baseline: 1.0327x; 1.0327x over previous
import functools

import jax
import jax.numpy as jnp
from jax import lax
from jax.experimental import pallas as pl
from jax.experimental.pallas import tpu as pltpu, tpu_sc as plsc

_BEAM = 20
_PRE = 30
_PAD = 32
_L = 16
_NEG = float("-inf")


@functools.cache
def _build():
    return functools.partial(
        pl.kernel,
        mesh=plsc.VectorSubcoreMesh(
            core_axis_name="c", subcore_axis_name="s", num_cores=1),
        out_type=[
            jax.ShapeDtypeStruct((_BEAM,), jnp.int32),
            jax.ShapeDtypeStruct((_BEAM,), jnp.int32),
        ],
        scratch_types=[
            pltpu.VMEM((_PAD,), jnp.int32),
            pltpu.VMEM((_PAD,), jnp.float32),
            pltpu.VMEM((_BEAM,), jnp.int32),
            pltpu.VMEM((_BEAM,), jnp.int32),
            pltpu.SemaphoreType.DMA,
            pltpu.SemaphoreType.DMA,
        ],
        compiler_params=pltpu.CompilerParams(needs_layout_passes=False),
    )(_beam_topk)


def _beam_topk(ws_hbm, ids_hbm, top_hbm, local_hbm,
               idx_v, vals_v, ot_v, ol_v, sem, sem2):
    is_worker = jnp.logical_and(
        lax.axis_index("c") == 0, lax.axis_index("s") == 0)

    @pl.when(is_worker)
    def _():
        idx_v[pl.ds(_L, _L)] = jnp.zeros((_L,), jnp.int32)
        pltpu.sync_copy(ids_hbm, idx_v.at[pl.ds(0, _PRE)])
        pltpu.async_copy(ws_hbm.at[idx_v], vals_v, sem).wait()

        lane = lax.iota(jnp.int32, _L)
        g0 = idx_v[pl.ds(0, _L)]
        g1 = idx_v[pl.ds(_L, _L)]
        v0 = vals_v[pl.ds(0, _L)]
        v1 = jnp.where(lane < _PRE - _L, vals_v[pl.ds(_L, _L)], _NEG)

        def bcast(a0, a1, j):
            src = a0 if j < _L else a1
            return jnp.take(src, jnp.full((_L,), j % _L, jnp.int32))

        false_v = jnp.zeros((_L,), jnp.bool_)
        dup0, dup1 = false_v, false_v
        for j in range(_PRE - 1):
            bg = bcast(g0, g1, j)
            if j < _L:
                dup0 = jnp.logical_or(
                    dup0, jnp.logical_and(g0 == bg, lane > j))
                dup1 = jnp.logical_or(dup1, g1 == bg)
            else:
                dup1 = jnp.logical_or(
                    dup1, jnp.logical_and(g1 == bg, lane > j - _L))
        neg = jnp.full((_L,), _NEG, jnp.float32)
        dval0 = jnp.where(dup0, neg, v0)
        dval1 = jnp.where(dup1, neg, v1)

        zero_v = jnp.zeros((_L,), jnp.int32)
        lrank0, lrank1 = zero_v, zero_v
        trank0, trank1 = zero_v, zero_v
        for j in range(_PRE):
            bv = bcast(v0, v1, j)
            bdv = bcast(dval0, dval1, j)
            bg = bcast(g0, g1, j)

            if j < _L:
                m0 = jnp.logical_or(
                    bv > v0, jnp.logical_and(bv == v0, lane > j))
                m1 = jnp.logical_or(bv > v1, bv == v1)
            else:
                m0 = bv > v0
                m1 = jnp.logical_or(
                    bv > v1, jnp.logical_and(bv == v1, lane > j - _L))
            lrank0 = lrank0 + m0.astype(jnp.int32)
            lrank1 = lrank1 + m1.astype(jnp.int32)

            t0 = jnp.logical_or(
                bdv > dval0, jnp.logical_and(bdv == dval0, bg < g0))
            t1 = jnp.logical_or(
                bdv > dval1, jnp.logical_and(bdv == dval1, bg < g1))
            trank0 = trank0 + t0.astype(jnp.int32)
            trank1 = trank1 + t1.astype(jnp.int32)

        beam1 = jnp.full((_L,), _BEAM - 1, jnp.int32)
        plsc.store_scatter(
            ot_v, [jnp.minimum(trank0, beam1)], g0, mask=trank0 < _BEAM)
        plsc.store_scatter(
            ot_v, [jnp.minimum(trank1, beam1)], g1, mask=trank1 < _BEAM)
        plsc.store_scatter(
            ol_v, [jnp.minimum(lrank0, beam1)], lane, mask=lrank0 < _BEAM)
        plsc.store_scatter(
            ol_v, [jnp.minimum(lrank1, beam1)], lane + _L,
            mask=lrank1 < _BEAM)

        c1 = pltpu.async_copy(ot_v, top_hbm, sem)
        c2 = pltpu.async_copy(ol_v, local_hbm, sem2)
        c1.wait()
        c2.wait()


def kernel(weighted_scores, ids):
    top_ids, local_ids = _build()(weighted_scores, ids)
    return top_ids, local_ids
